# baseline (speedup 1.0000x reference)
import functools

import jax
import jax.numpy as jnp
from jax import lax
from jax.experimental import pallas as pl
from jax.experimental.pallas import tpu as pltpu
from jax.experimental.pallas import tpu_sc as plsc

_NUM_EXPERTS = 64
_TOP_K = 8
_NEG = -1e30


def _gemm_block(x_ref, w_ref, logits_ref, logits_t_ref):
    x = x_ref[...]
    w = w_ref[...]
    logits_ref[...] = lax.dot_general(
        x, w, (((1,), (1,)), ((), ())), preferred_element_type=jnp.float32,
    )
    logits_t_ref[...] = lax.dot_general(
        w, x, (((1,), (1,)), ((), ())), preferred_element_type=jnp.float32,
    )


def _router_gemm(x, w, block_t):
    n, d = x.shape
    e = w.shape[0]
    return pl.pallas_call(
        _gemm_block,
        grid=(n // block_t,),
        in_specs=[
            pl.BlockSpec((block_t, d), lambda i: (i, 0)),
            pl.BlockSpec((e, d), lambda i: (0, 0)),
        ],
        out_specs=[
            pl.BlockSpec((block_t, e), lambda i: (i, 0)),
            pl.BlockSpec((e, block_t), lambda i: (0, i)),
        ],
        out_shape=[
            jax.ShapeDtypeStruct((n, e), jnp.float32),
            jax.ShapeDtypeStruct((e, n), jnp.float32),
        ],
        compiler_params=pltpu.CompilerParams(
            dimension_semantics=("arbitrary",),
        ),
    )(x, w)


def _make_sc_route(n):
    info = plsc.get_sparse_core_info()
    nw = info.num_cores * info.num_subcores
    toks_per_w = n // nw
    n_groups = toks_per_w // 16
    mesh = plsc.VectorSubcoreMesh(core_axis_name="c", subcore_axis_name="s")

    @functools.partial(
        pl.kernel,
        mesh=mesh,
        out_type=[
            jax.ShapeDtypeStruct((_TOP_K, n), jnp.float32),
            jax.ShapeDtypeStruct((_TOP_K, n), jnp.int32),
        ],
        scratch_types=[
            pltpu.VMEM((_NUM_EXPERTS, toks_per_w), jnp.float32),
            pltpu.VMEM((_TOP_K, toks_per_w), jnp.float32),
            pltpu.VMEM((_TOP_K, toks_per_w), jnp.int32),
        ],
    )
    def sc_route(lt_hbm, wout_hbm, iout_hbm, buf, wbuf, ibuf):
        wid = lax.axis_index("s") * info.num_cores + lax.axis_index("c")
        tok0 = wid * toks_per_w
        pltpu.sync_copy(lt_hbm.at[:, pl.ds(tok0, toks_per_w)], buf)

        cvecs = [jnp.full((16,), e, jnp.int32) for e in range(_NUM_EXPERTS)]

        def body(g, carry):
            t0 = g * 16
            work = [buf[e, pl.ds(t0, 16)] for e in range(_NUM_EXPERTS)]
            rank_v = []
            rank_i = []
            for _ in range(_TOP_K):
                level = work
                while len(level) > 1:
                    level = [
                        jnp.maximum(level[2 * i], level[2 * i + 1])
                        for i in range(len(level) // 2)
                    ]
                m = level[0]
                idx = cvecs[0]
                for e in range(_NUM_EXPERTS - 1, -1, -1):
                    eq = work[e] == m
                    idx = jnp.where(eq, cvecs[e], idx)
                    work[e] = jnp.where(eq, _NEG, work[e])
                rank_v.append(m)
                rank_i.append(idx)
            exps = [jnp.exp(v - rank_v[0]) for v in rank_v]
            s = exps[0]
            for e_ in exps[1:]:
                s = s + e_
            inv = 1.0 / s
            for j in range(_TOP_K):
                wbuf[j, pl.ds(t0, 16)] = exps[j] * inv
                ibuf[j, pl.ds(t0, 16)] = rank_i[j]
            return carry

        lax.fori_loop(0, n_groups, body, 0)

        cols = pl.ds(tok0, toks_per_w)
        pltpu.sync_copy(wbuf, wout_hbm.at[:, cols])
        pltpu.sync_copy(ibuf, iout_hbm.at[:, cols])

    return sc_route


@jax.jit
def kernel(hidden_states, W):
    n = hidden_states.shape[0]
    block_t = min(1024, n)
    logits, logits_t = _router_gemm(hidden_states, W, block_t)
    wout, iout = _make_sc_route(n)(logits_t)
    return (wout.T, logits, iout.T)


# speedup vs baseline: 1.0969x; 1.0969x over previous
import functools

import jax
import jax.numpy as jnp
from jax import lax
from jax.experimental import pallas as pl
from jax.experimental.pallas import tpu as pltpu

_NUM_EXPERTS = 64
_TOP_K = 8
_NEG = -1e30


def _router_block(x_ref, w_ref, logits_ref, weights_ref, idx_ref):
    w = w_ref[...]
    t_full = x_ref.shape[0]
    n_sub = 8
    t_sub = t_full // n_sub
    sub_logits = []
    for s in range(n_sub):
        rows = pl.ds(s * t_sub, t_sub)
        logits = lax.dot_general(
            x_ref[rows, :], w, (((1,), (1,)), ((), ())),
            preferred_element_type=jnp.float32,
        )
        logits_ref[rows, :] = logits
        sub_logits.append(logits)
    for s in range(n_sub):
        rows = pl.ds(s * t_sub, t_sub)
        wgt, idx = _topk_epilogue(sub_logits[s])
        weights_ref[rows, :] = wgt
        idx_ref[rows, :] = idx


def _topk_epilogue(logits):
    t = logits.shape[0]
    e_dim = logits.shape[1]
    iota_f = lax.broadcasted_iota(jnp.int32, (t, e_dim), 1).astype(jnp.float32)
    sentinel = float(e_dim)
    work = logits
    vals = []
    idxs = []
    for _ in range(_TOP_K):
        m = jnp.max(work, axis=1, keepdims=True)
        mask = work == m
        idx = jnp.min(jnp.where(mask, iota_f, sentinel), axis=1, keepdims=True)
        vals.append(m)
        idxs.append(idx)
        work = jnp.where(mask, _NEG, work)
    topv = jnp.concatenate(vals, axis=1)
    topi = jnp.concatenate(idxs, axis=1)
    ex = jnp.exp(topv - topv[:, 0:1])
    return ex / jnp.sum(ex, axis=1, keepdims=True), topi.astype(jnp.int32)


def _route(x, w, block_t):
    n, d = x.shape
    e = w.shape[0]
    grid = (n // block_t,)
    return pl.pallas_call(
        _router_block,
        grid=grid,
        in_specs=[
            pl.BlockSpec((block_t, d), lambda i: (i, 0)),
            pl.BlockSpec((e, d), lambda i: (0, 0)),
        ],
        out_specs=[
            pl.BlockSpec((block_t, e), lambda i: (i, 0)),
            pl.BlockSpec((block_t, _TOP_K), lambda i: (i, 0)),
            pl.BlockSpec((block_t, _TOP_K), lambda i: (i, 0)),
        ],
        out_shape=[
            jax.ShapeDtypeStruct((n, e), jnp.float32),
            jax.ShapeDtypeStruct((n, _TOP_K), jnp.float32),
            jax.ShapeDtypeStruct((n, _TOP_K), jnp.int32),
        ],
        compiler_params=pltpu.CompilerParams(
            dimension_semantics=("arbitrary",),
        ),
    )(x, w)


@jax.jit
def kernel(hidden_states, W):
    n = hidden_states.shape[0]
    block_t = min(1024, n)
    logits, topk_weight, topk_idx = _route(hidden_states, W, block_t)
    return (topk_weight, logits, topk_idx)
